# baseline (speedup 1.0000x reference)
import functools

import jax
import jax.numpy as jnp
from jax.experimental import pallas as pl
from jax.experimental.pallas import tpu as pltpu

SCALES = (128.0, 256.0, 512.0)
RATIOS = (0.5, 1.0, 2.0)
K = 9
NMS_TH = 0.7
PRE_TOPK = 10000
POST_TOPK = 2000
MIN_SIZE = 16.0

_GH = 50
_GW = 50
_PW = 56
_NP = _PW * _PW
_BLK = 128
_NBLK = 22
_ROWS_OUT = _NBLK * _BLK
_ROWS_IN = _ROWS_OUT + _BLK


def _conv_body(x0_ref, x1_ref, x2_ref, w9_ref, br_ref, wh_ref, bh_ref, out_ref):
    i = pl.program_id(0)
    base = i * _BLK
    xrefs = (x0_ref, x1_ref, x2_ref)
    acc = jnp.zeros((_BLK, 512), dtype=jnp.float32)
    for k in range(9):
        dy, dx = k // 3, k % 3
        xk = xrefs[dx][pl.ds(base + dy * _PW, _BLK), :]
        acc = acc + jnp.dot(xk, w9_ref[k], preferred_element_type=jnp.float32)
    hidden = jnp.maximum(acc + br_ref[0, :], 0.0)
    head = jnp.dot(hidden, wh_ref[:, :], preferred_element_type=jnp.float32)
    out_ref[...] = head + bh_ref[0, :]


def _conv_heads(feat, W_rpn, b_rpn, W_cls, b_cls, W_reg, b_reg):
    x = jnp.transpose(feat[0], (1, 2, 0))
    xp = jnp.zeros((_PW, _PW, 512), jnp.float32).at[1:51, 1:51, :].set(x)
    xp = xp.reshape(_NP, 512)
    xpf = jnp.zeros((_ROWS_IN + 8, 512), jnp.float32).at[:_ROWS_IN, :].set(xp[:_ROWS_IN])
    xs = [xpf[dx:dx + _ROWS_IN, :] for dx in range(3)]

    w9 = jnp.transpose(W_rpn, (2, 3, 1, 0)).reshape(9, 512, 512)
    wc = W_cls[:, :, 0, 0].T
    wr = W_reg[:, :, 0, 0].T
    wh = jnp.zeros((512, 128), jnp.float32)
    wh = wh.at[:, :9].set(wc).at[:, 9:45].set(wr)
    bh = jnp.zeros((1, 128), jnp.float32)
    bh = bh.at[0, :9].set(b_cls).at[0, 9:45].set(b_reg)
    br = b_rpn.reshape(1, 512)

    out = pl.pallas_call(
        _conv_body,
        grid=(_NBLK,),
        in_specs=[
            pl.BlockSpec((_ROWS_IN, 512), lambda i: (0, 0)),
            pl.BlockSpec((_ROWS_IN, 512), lambda i: (0, 0)),
            pl.BlockSpec((_ROWS_IN, 512), lambda i: (0, 0)),
            pl.BlockSpec((9, 512, 512), lambda i: (0, 0, 0)),
            pl.BlockSpec((1, 512), lambda i: (0, 0)),
            pl.BlockSpec((512, 128), lambda i: (0, 0)),
            pl.BlockSpec((1, 128), lambda i: (0, 0)),
        ],
        out_specs=pl.BlockSpec((_BLK, 128), lambda i: (i, 0)),
        out_shape=jax.ShapeDtypeStruct((_ROWS_OUT, 128), jnp.float32),
    )(xs[0], xs[1], xs[2], w9, br, wh, bh)

    grid_out = (
        jnp.zeros((_NP, 128), jnp.float32).at[:_ROWS_OUT, :].set(out)
        .reshape(_PW, _PW, 128)[:_GH, :_GW, :])
    cls_flat = grid_out[:, :, :9].reshape(-1)
    reg_flat = grid_out[:, :, 9:45].reshape(-1, 4)
    return cls_flat, reg_flat


def _anchor_geometry():
    scales = jnp.array(SCALES, dtype=jnp.float32)
    ratios = jnp.array(RATIOS, dtype=jnp.float32)
    h_ratio = jnp.sqrt(ratios)
    w_ratio = 1.0 / h_ratio
    ws = (w_ratio[:, None] * scales[None, :]).reshape(-1)
    hs = (h_ratio[:, None] * scales[None, :]).reshape(-1)
    base = jnp.round(jnp.stack([-ws, -hs, ws, hs], axis=1) / 2.0)
    bw = base[:, 2] - base[:, 0]
    bh = base[:, 3] - base[:, 1]
    bcx = base[:, 0] + 0.5 * bw
    bcy = base[:, 1] + 0.5 * bh
    sx = jnp.arange(_GW, dtype=jnp.float32)
    sy = jnp.arange(_GH, dtype=jnp.float32)
    syy, sxx = jnp.meshgrid(sy, sx, indexing="ij")
    sxx = sxx.reshape(-1)
    syy = syy.reshape(-1)
    aw = jnp.broadcast_to(bw[None, :], (_GH * _GW, K)).reshape(-1)
    ah = jnp.broadcast_to(bh[None, :], (_GH * _GW, K)).reshape(-1)
    acx = (sxx[:, None] + bcx[None, :]).reshape(-1)
    acy = (syy[:, None] + bcy[None, :]).reshape(-1)
    return aw, ah, acx, acy


def _nms_keep_mask(boxes, scores, iou_th):
    n = boxes.shape[0]
    order = jnp.argsort(-scores)
    b = boxes[order]
    x1, y1, x2, y2 = b[:, 0], b[:, 1], b[:, 2], b[:, 3]
    areas = (x2 - x1) * (y2 - y1)
    idx = jnp.arange(n)

    def body(i, suppressed):
        active = jnp.logical_not(suppressed[i])
        xx1 = jnp.maximum(x1[i], x1)
        yy1 = jnp.maximum(y1[i], y1)
        xx2 = jnp.minimum(x2[i], x2)
        yy2 = jnp.minimum(y2[i], y2)
        inter = jnp.maximum(xx2 - xx1, 0.0) * jnp.maximum(yy2 - yy1, 0.0)
        iou = inter / (areas[i] + areas - inter + 1e-9)
        return suppressed | (active & (iou > iou_th) & (idx > i))

    suppressed = jax.lax.fori_loop(0, n, body, jnp.zeros((n,), dtype=bool))
    return jnp.zeros((n,), dtype=bool).at[order].set(jnp.logical_not(suppressed))


def kernel(feat, image, W_rpn, b_rpn, W_cls, b_cls, W_reg, b_reg):
    img_h, img_w = image.shape[-2], image.shape[-1]
    cls_flat, reg_flat = _conv_heads(feat, W_rpn, b_rpn, W_cls, b_cls, W_reg, b_reg)

    aw, ah, acx, acy = _anchor_geometry()
    dx, dy, dw, dh = reg_flat[:, 0], reg_flat[:, 1], reg_flat[:, 2], reg_flat[:, 3]
    pcx = dx * aw + acx
    pcy = dy * ah + acy
    pw = jnp.exp(dw) * aw
    ph = jnp.exp(dh) * ah
    proposals = jnp.stack(
        [pcx - 0.5 * pw, pcy - 0.5 * ph, pcx + 0.5 * pw, pcy + 0.5 * ph], axis=1)

    scores = jax.nn.sigmoid(cls_flat)
    _, top_idx = jax.lax.top_k(scores, PRE_TOPK)
    scores_t = scores[top_idx]
    props = proposals[top_idx]
    px1 = jnp.clip(props[:, 0], 0.0, float(img_w))
    py1 = jnp.clip(props[:, 1], 0.0, float(img_h))
    px2 = jnp.clip(props[:, 2], 0.0, float(img_w))
    py2 = jnp.clip(props[:, 3], 0.0, float(img_h))
    props = jnp.stack([px1, py1, px2, py2], axis=1)
    ws = props[:, 2] - props[:, 0]
    hs = props[:, 3] - props[:, 1]
    valid = (ws >= MIN_SIZE) & (hs >= MIN_SIZE)
    masked = jnp.where(valid, scores_t, -1.0)
    keep = _nms_keep_mask(props, masked, NMS_TH) & valid
    final_masked = jnp.where(keep, masked, -1.0)
    _, post_idx = jax.lax.top_k(final_masked, POST_TOPK)
    keep_g = keep[post_idx]
    out_props = jnp.where(keep_g[:, None], props[post_idx], 0.0)
    out_scores = jnp.where(keep_g, scores_t[post_idx], 0.0)
    return out_props, out_scores


# speedup vs baseline: 35.5260x; 35.5260x over previous
import functools

import jax
import jax.numpy as jnp
from jax.experimental import pallas as pl
from jax.experimental.pallas import tpu as pltpu

SCALES = (128.0, 256.0, 512.0)
RATIOS = (0.5, 1.0, 2.0)
K = 9
NMS_TH = 0.7
PRE_TOPK = 10000
POST_TOPK = 2000
MIN_SIZE = 16.0

_GH = 50
_GW = 50
_PW = 56
_NP = _PW * _PW
_BLK = 128
_NBLK = 22
_ROWS_OUT = _NBLK * _BLK
_ROWS_IN = _ROWS_OUT + _BLK


def _conv_body(x0_ref, x1_ref, x2_ref, w9_ref, br_ref, wh_ref, bh_ref, out_ref):
    i = pl.program_id(0)
    base = i * _BLK
    xrefs = (x0_ref, x1_ref, x2_ref)
    acc = jnp.zeros((_BLK, 512), dtype=jnp.float32)
    for k in range(9):
        dy, dx = k // 3, k % 3
        xk = xrefs[dx][pl.ds(base + dy * _PW, _BLK), :]
        acc = acc + jnp.dot(xk, w9_ref[k], preferred_element_type=jnp.float32)
    hidden = jnp.maximum(acc + br_ref[0, :], 0.0)
    head = jnp.dot(hidden, wh_ref[:, :], preferred_element_type=jnp.float32)
    out_ref[...] = head + bh_ref[0, :]


def _conv_heads(feat, W_rpn, b_rpn, W_cls, b_cls, W_reg, b_reg):
    x = jnp.transpose(feat[0], (1, 2, 0))
    xp = jnp.zeros((_PW, _PW, 512), jnp.float32).at[1:51, 1:51, :].set(x)
    xp = xp.reshape(_NP, 512)
    xpf = jnp.zeros((_ROWS_IN + 8, 512), jnp.float32).at[:_ROWS_IN, :].set(xp[:_ROWS_IN])
    xs = [xpf[dx:dx + _ROWS_IN, :] for dx in range(3)]

    w9 = jnp.transpose(W_rpn, (2, 3, 1, 0)).reshape(9, 512, 512)
    wc = W_cls[:, :, 0, 0].T
    wr = W_reg[:, :, 0, 0].T
    wh = jnp.zeros((512, 128), jnp.float32)
    wh = wh.at[:, :9].set(wc).at[:, 9:45].set(wr)
    bh = jnp.zeros((1, 128), jnp.float32)
    bh = bh.at[0, :9].set(b_cls).at[0, 9:45].set(b_reg)
    br = b_rpn.reshape(1, 512)

    out = pl.pallas_call(
        _conv_body,
        grid=(_NBLK,),
        in_specs=[
            pl.BlockSpec((_ROWS_IN, 512), lambda i: (0, 0)),
            pl.BlockSpec((_ROWS_IN, 512), lambda i: (0, 0)),
            pl.BlockSpec((_ROWS_IN, 512), lambda i: (0, 0)),
            pl.BlockSpec((9, 512, 512), lambda i: (0, 0, 0)),
            pl.BlockSpec((1, 512), lambda i: (0, 0)),
            pl.BlockSpec((512, 128), lambda i: (0, 0)),
            pl.BlockSpec((1, 128), lambda i: (0, 0)),
        ],
        out_specs=pl.BlockSpec((_BLK, 128), lambda i: (i, 0)),
        out_shape=jax.ShapeDtypeStruct((_ROWS_OUT, 128), jnp.float32),
    )(xs[0], xs[1], xs[2], w9, br, wh, bh)

    grid_out = (
        jnp.zeros((_NP, 128), jnp.float32).at[:_ROWS_OUT, :].set(out)
        .reshape(_PW, _PW, 128)[:_GH, :_GW, :])
    cls_flat = grid_out[:, :, :9].reshape(-1)
    reg_flat = grid_out[:, :, 9:45].reshape(-1, 4)
    return cls_flat, reg_flat


def _anchor_geometry():
    scales = jnp.array(SCALES, dtype=jnp.float32)
    ratios = jnp.array(RATIOS, dtype=jnp.float32)
    h_ratio = jnp.sqrt(ratios)
    w_ratio = 1.0 / h_ratio
    ws = (w_ratio[:, None] * scales[None, :]).reshape(-1)
    hs = (h_ratio[:, None] * scales[None, :]).reshape(-1)
    base = jnp.round(jnp.stack([-ws, -hs, ws, hs], axis=1) / 2.0)
    bw = base[:, 2] - base[:, 0]
    bh = base[:, 3] - base[:, 1]
    bcx = base[:, 0] + 0.5 * bw
    bcy = base[:, 1] + 0.5 * bh
    sx = jnp.arange(_GW, dtype=jnp.float32)
    sy = jnp.arange(_GH, dtype=jnp.float32)
    syy, sxx = jnp.meshgrid(sy, sx, indexing="ij")
    sxx = sxx.reshape(-1)
    syy = syy.reshape(-1)
    aw = jnp.broadcast_to(bw[None, :], (_GH * _GW, K)).reshape(-1)
    ah = jnp.broadcast_to(bh[None, :], (_GH * _GW, K)).reshape(-1)
    acx = (sxx[:, None] + bcx[None, :]).reshape(-1)
    acy = (syy[:, None] + bcy[None, :]).reshape(-1)
    return aw, ah, acx, acy


_NMS_N = 10240
_NMS_NB = _NMS_N // _BLK


def _iou_gt(x1c, y1c, x2c, y2c, ac, x1j, y1j, x2j, y2j, aj):
    xx1 = jnp.maximum(x1c, x1j)
    yy1 = jnp.maximum(y1c, y1j)
    xx2 = jnp.minimum(x2c, x2j)
    yy2 = jnp.minimum(y2c, y2j)
    inter = jnp.maximum(xx2 - xx1, 0.0) * jnp.maximum(yy2 - yy1, 0.0)
    iou = inter / (ac + aj - inter + 1e-9)
    return (iou > NMS_TH).astype(jnp.float32)


def _nms_body(colmat_ref, x1r_ref, y1r_ref, x2r_ref, y2r_ref, ar_ref,
              validr_ref, keep_ref, sup_ref, m_ref):
    bi = pl.program_id(0)

    @pl.when(bi == 0)
    def _init():
        sup_ref[...] = 1.0 - validr_ref[...]

    cb = colmat_ref[pl.ds(bi * _BLK, _BLK), :]
    x1c, y1c = cb[:, 0:1], cb[:, 1:2]
    x2c, y2c = cb[:, 2:3], cb[:, 3:4]
    ac = cb[:, 4:5]

    m = _iou_gt(x1c, y1c, x2c, y2c, ac,
                x1r_ref[bi], y1r_ref[bi], x2r_ref[bi], y2r_ref[bi], ar_ref[bi])
    ii = jax.lax.broadcasted_iota(jnp.int32, (_BLK, _BLK), 0)
    jj = jax.lax.broadcasted_iota(jnp.int32, (_BLK, _BLK), 1)
    m = jnp.where(jj > ii, m, 0.0)
    m_ref[...] = m[:, None, :]

    lidx = jax.lax.broadcasted_iota(jnp.int32, (1, _BLK), 1)
    sup0 = sup_ref[bi]

    def intra(r, sup):
        srv = jnp.sum(jnp.where(lidx == r, sup, 0.0))
        return jnp.where(srv < 0.5, jnp.maximum(sup, m_ref[r]), sup)

    sup = jax.lax.fori_loop(0, _BLK, intra, sup0)
    sup_ref[bi] = sup
    keep_ref[...] = ((1.0 - sup) * validr_ref[bi])[None]
    kept = 1.0 - sup

    def cross(cj, _):
        m2 = _iou_gt(x1c, y1c, x2c, y2c, ac,
                     x1r_ref[cj], y1r_ref[cj], x2r_ref[cj], y2r_ref[cj],
                     ar_ref[cj])
        hits = jnp.dot(kept, m2, preferred_element_type=jnp.float32)
        sup_ref[cj] = jnp.maximum(sup_ref[cj], (hits > 0.0).astype(jnp.float32))
        return 0

    jax.lax.fori_loop(bi + 1, _NMS_NB, cross, 0)


def _nms_keep_pallas(x1, y1, x2, y2, valid):
    n = x1.shape[0]
    area = (x2 - x1) * (y2 - y1)
    pad = _NMS_N - n

    def padded(v):
        return jnp.concatenate([v, jnp.zeros((pad,), v.dtype)])

    x1p, y1p, x2p, y2p, ap = (padded(v) for v in (x1, y1, x2, y2, area))
    vp = padded(valid.astype(jnp.float32))
    colmat = jnp.zeros((_NMS_N, 8), jnp.float32)
    colmat = (colmat.at[:, 0].set(x1p).at[:, 1].set(y1p)
              .at[:, 2].set(x2p).at[:, 3].set(y2p).at[:, 4].set(ap))
    rows = [v.reshape(_NMS_NB, 1, _BLK) for v in (x1p, y1p, x2p, y2p, ap, vp)]

    keep = pl.pallas_call(
        _nms_body,
        grid=(_NMS_NB,),
        in_specs=[pl.BlockSpec((_NMS_N, 8), lambda i: (0, 0))]
        + [pl.BlockSpec((_NMS_NB, 1, _BLK), lambda i: (0, 0, 0))] * 6,
        out_specs=pl.BlockSpec((1, 1, _BLK), lambda i: (i, 0, 0)),
        out_shape=jax.ShapeDtypeStruct((_NMS_NB, 1, _BLK), jnp.float32),
        scratch_shapes=[
            pltpu.VMEM((_NMS_NB, 1, _BLK), jnp.float32),
            pltpu.VMEM((_BLK, 1, _BLK), jnp.float32),
        ],
    )(colmat, *rows)
    return keep.reshape(_NMS_N)[:n] > 0.5


def kernel(feat, image, W_rpn, b_rpn, W_cls, b_cls, W_reg, b_reg):
    img_h, img_w = image.shape[-2], image.shape[-1]
    cls_flat, reg_flat = _conv_heads(feat, W_rpn, b_rpn, W_cls, b_cls, W_reg, b_reg)

    aw, ah, acx, acy = _anchor_geometry()
    dx, dy, dw, dh = reg_flat[:, 0], reg_flat[:, 1], reg_flat[:, 2], reg_flat[:, 3]
    pcx = dx * aw + acx
    pcy = dy * ah + acy
    pw = jnp.exp(dw) * aw
    ph = jnp.exp(dh) * ah
    proposals = jnp.stack(
        [pcx - 0.5 * pw, pcy - 0.5 * ph, pcx + 0.5 * pw, pcy + 0.5 * ph], axis=1)

    scores = jax.nn.sigmoid(cls_flat)
    _, top_idx = jax.lax.top_k(scores, PRE_TOPK)
    scores_t = scores[top_idx]
    props = proposals[top_idx]
    px1 = jnp.clip(props[:, 0], 0.0, float(img_w))
    py1 = jnp.clip(props[:, 1], 0.0, float(img_h))
    px2 = jnp.clip(props[:, 2], 0.0, float(img_w))
    py2 = jnp.clip(props[:, 3], 0.0, float(img_h))
    props = jnp.stack([px1, py1, px2, py2], axis=1)
    ws = props[:, 2] - props[:, 0]
    hs = props[:, 3] - props[:, 1]
    valid = (ws >= MIN_SIZE) & (hs >= MIN_SIZE)
    masked = jnp.where(valid, scores_t, -1.0)
    keep = _nms_keep_pallas(px1, py1, px2, py2, valid)
    final_masked = jnp.where(keep, masked, -1.0)
    _, post_idx = jax.lax.top_k(final_masked, POST_TOPK)
    keep_g = keep[post_idx]
    out_props = jnp.where(keep_g[:, None], props[post_idx], 0.0)
    out_scores = jnp.where(keep_g, scores_t[post_idx], 0.0)
    return out_props, out_scores
